# Initial kernel scaffold; baseline (speedup 1.0000x reference)
import jax
import jax.numpy as jnp
from jax.experimental import pallas as pl


def kernel(x, edge_index, batch_index, W1, b1, W2, b2, W3, b3, Wa, ba):
    raise NotImplementedError("write your pallas kernel here")


# speedup vs baseline: 21.8760x; 21.8760x over previous
import functools
import jax
import jax.numpy as jnp
from jax import lax
from jax.experimental import pallas as pl
from jax.experimental.pallas import tpu as pltpu
from jax.experimental.pallas import tpu_sc as plsc

N = 10000
E = 320000
F = 128
HD = 128
CO = 32
G = 64
NC = 2
NS = 16
NW = NC * NS
CHUNK = 80
ROWS = E // CHUNK
RPT = ROWS // NW
NPAD = 640
PPAD = 320

_f32 = jnp.float32
_mesh = plsc.VectorSubcoreMesh(core_axis_name="c", subcore_axis_name="s")


def _nj16(s):
    return jnp.where(s < NS - 1, NPAD // CHUNK, (N - NPAD * (NS - 1)) // CHUNK)


def _nj32(w):
    return jnp.where(w < NW - 1, PPAD // CHUNK, (N - PPAD * (NW - 1)) // CHUNK)


@functools.partial(
    pl.kernel,
    out_type=jax.ShapeDtypeStruct((NC, N, HD), _f32),
    mesh=_mesh,
    scratch_types=[
        pltpu.VMEM((RPT, CHUNK), jnp.int32),
        pltpu.VMEM((CHUNK, HD), _f32),
        pltpu.VMEM_SHARED((N, HD), _f32),
    ],
)
def _deg_kernel(dst_hbm, deg_hbm, dstv, onesb, acc1):
    c = lax.axis_index("c")
    s = lax.axis_index("s")
    w = s * NC + c
    nj = _nj16(s)

    def fill(val):
        def fill_body(i, _):
            for k in range(HD // 16):
                onesb[i, pl.ds(k * 16, 16)] = jnp.full((16,), val, _f32)
            return 0

        lax.fori_loop(0, CHUNK, fill_body, 0)

    fill(0.0)

    def zacc_body(j, _):
        pltpu.sync_copy(onesb, acc1.at[pl.ds(NPAD * s + CHUNK * j, CHUNK)])
        return 0

    lax.fori_loop(0, nj, zacc_body, 0)
    fill(1.0)
    plsc.subcore_barrier()

    pltpu.sync_copy(dst_hbm.at[w], dstv)

    def count_body(j, _):
        pltpu.sync_copy(onesb, acc1.at[dstv.at[j]], add=True)
        return 0

    lax.fori_loop(0, RPT, count_body, 0)
    plsc.subcore_barrier()

    def dump_body(j, _):
        r0 = NPAD * s + CHUNK * j
        pltpu.sync_copy(acc1.at[pl.ds(r0, CHUNK)],
                        deg_hbm.at[c].at[pl.ds(r0, CHUNK)])
        return 0

    lax.fori_loop(0, nj, dump_body, 0)


def _tc0_body(deg_ref, x_ref, w_ref, z_ref, dinv_ref):
    d3 = deg_ref[...]
    deg = (jnp.max(d3[0], axis=1, keepdims=True)
           + jnp.max(d3[1], axis=1, keepdims=True) + 1.0)
    dinv = lax.rsqrt(deg)
    z = jnp.dot(x_ref[...], w_ref[...], preferred_element_type=_f32)
    z_ref[...] = z * dinv
    dinv_ref[...] = dinv


def _tc0(deg16, x, w):
    r = 2000
    return pl.pallas_call(
        _tc0_body,
        grid=(N // r,),
        in_specs=[
            pl.BlockSpec((NC, r, HD), lambda i: (0, i, 0)),
            pl.BlockSpec((r, F), lambda i: (i, 0)),
            pl.BlockSpec((F, HD), lambda i: (0, 0)),
        ],
        out_specs=[
            pl.BlockSpec((r, HD), lambda i: (i, 0)),
            pl.BlockSpec((r, 1), lambda i: (i, 0)),
        ],
        out_shape=[
            jax.ShapeDtypeStruct((N, HD), _f32),
            jax.ShapeDtypeStruct((N, 1), _f32),
        ],
    )(deg16, x, w)


def _tcmid_body(acc_ref, z_ref, dinv_ref, b_ref, w_ref, zo_ref):
    a3 = acc_ref[...]
    dinv = dinv_ref[...]
    h = (a3[0] + a3[1] + z_ref[...]) * dinv + b_ref[...]
    h = jnp.maximum(h, 0.0)
    zo = jnp.dot(h, w_ref[...], preferred_element_type=_f32)
    zo_ref[...] = zo * dinv


def _tcmid(acc, z, dinv, b, w):
    r = 2000
    return pl.pallas_call(
        _tcmid_body,
        grid=(N // r,),
        in_specs=[
            pl.BlockSpec((NC, r, HD), lambda i: (0, i, 0)),
            pl.BlockSpec((r, HD), lambda i: (i, 0)),
            pl.BlockSpec((r, 1), lambda i: (i, 0)),
            pl.BlockSpec((1, HD), lambda i: (0, 0)),
            pl.BlockSpec((HD, HD), lambda i: (0, 0)),
        ],
        out_specs=pl.BlockSpec((r, HD), lambda i: (i, 0)),
        out_shape=jax.ShapeDtypeStruct((N, HD), _f32),
    )(acc, z, dinv, b.reshape(1, HD), w)


@functools.partial(
    pl.kernel,
    out_type=jax.ShapeDtypeStruct((NC, N, HD), _f32),
    mesh=_mesh,
    scratch_types=[
        pltpu.VMEM((25, CHUNK), jnp.int32),
        pltpu.VMEM((25, CHUNK), jnp.int32),
        pltpu.VMEM((CHUNK, HD), _f32),
        pltpu.VMEM((CHUNK, HD), _f32),
        pltpu.VMEM_SHARED((N, HD), _f32),
        pltpu.SemaphoreType.DMA,
        pltpu.SemaphoreType.DMA,
    ],
)
def _sc_scatter(src_hbm, dst_hbm, z_hbm, acc_hbm,
                srcv, dstv, buf0, buf1, acc, sem0, sem1):
    c = lax.axis_index("c")
    s = lax.axis_index("s")
    w = s * NC + c
    nj = _nj16(s)
    zeros = jnp.zeros((16,), _f32)

    def zrow_body(i, _):
        for k in range(HD // 16):
            buf0[i, pl.ds(k * 16, 16)] = zeros
        return 0

    lax.fori_loop(0, CHUNK, zrow_body, 0)

    def zacc_body(j, _):
        pltpu.sync_copy(buf0, acc.at[pl.ds(NPAD * s + CHUNK * j, CHUNK)])
        return 0

    lax.fori_loop(0, nj, zacc_body, 0)
    plsc.subcore_barrier()

    def gather(row, buf, sem):
        pltpu.make_async_copy(z_hbm.at[srcv.at[row]], buf, sem).start()

    def drain(row, buf, sem):
        pltpu.make_async_copy(z_hbm.at[srcv.at[row]], buf, sem).wait()

    def scat(row, buf):
        pltpu.sync_copy(buf, acc.at[dstv.at[row]], add=True)

    def block_body(oj, _):
        pltpu.sync_copy(src_hbm.at[w, oj], srcv)
        pltpu.sync_copy(dst_hbm.at[w, oj], dstv)
        gather(0, buf0, sem0)

        def pair_body(jj, _):
            j0 = 2 * jj
            gather(j0 + 1, buf1, sem1)
            drain(j0, buf0, sem0)
            scat(j0, buf0)
            gather(j0 + 2, buf0, sem0)
            drain(j0 + 1, buf1, sem1)
            scat(j0 + 1, buf1)
            return 0

        lax.fori_loop(0, 12, pair_body, 0)
        drain(24, buf0, sem0)
        scat(24, buf0)
        return 0

    lax.fori_loop(0, RPT // 25, block_body, 0)
    plsc.subcore_barrier()

    def dump_body(j, _):
        r0 = NPAD * s + CHUNK * j
        pltpu.sync_copy(acc.at[pl.ds(r0, CHUNK)],
                        acc_hbm.at[c].at[pl.ds(r0, CHUNK)])
        return 0

    lax.fori_loop(0, nj, dump_body, 0)


@functools.partial(
    pl.kernel,
    out_type=[
        jax.ShapeDtypeStruct((NC, NS, G, HD), _f32),
        jax.ShapeDtypeStruct((NC, NS, G, HD), _f32),
        jax.ShapeDtypeStruct((NC, NS, G, HD), _f32),
    ],
    mesh=_mesh,
    scratch_types=[
        pltpu.VMEM((CHUNK, HD), _f32),
        pltpu.VMEM((CHUNK, HD), _f32),
        pltpu.VMEM((CHUNK, HD), _f32),
        pltpu.VMEM((CHUNK + 16,), _f32),
        pltpu.VMEM((CHUNK + 16,), jnp.int32),
        pltpu.VMEM((HD,), _f32),
        pltpu.VMEM((G, HD), _f32),
        pltpu.VMEM((G, HD), _f32),
        pltpu.VMEM((G, HD), _f32),
    ],
)
def _sc_pool(acc_hbm, z_hbm, dinv_hbm, bidx_hbm, bias_hbm,
             psum_hbm, pmax_hbm, pcnt_hbm,
             arow0, arow1, zrow, dinvv, bidxv, biasv, psum, pmax, pcnt):
    c = lax.axis_index("c")
    s = lax.axis_index("s")
    w = s * NC + c
    nj = _nj32(w)
    zeros = jnp.zeros((16,), _f32)
    ones = jnp.ones((16,), _f32)
    neginf = jnp.full((16,), -jnp.inf, _f32)

    def pzero_body(g, _):
        for k in range(HD // 16):
            sl = pl.ds(k * 16, 16)
            psum[g, sl] = zeros
            pmax[g, sl] = neginf
            pcnt[g, sl] = zeros
        return 0

    lax.fori_loop(0, G, pzero_body, 0)
    pltpu.sync_copy(bias_hbm, biasv)

    def chunk_body(j, _):
        r0 = PPAD * w + CHUNK * j
        pltpu.sync_copy(acc_hbm.at[0].at[pl.ds(r0, CHUNK)], arow0)
        pltpu.sync_copy(acc_hbm.at[1].at[pl.ds(r0, CHUNK)], arow1)
        pltpu.sync_copy(z_hbm.at[pl.ds(r0, CHUNK)], zrow)
        pltpu.sync_copy(dinv_hbm.at[pl.ds(r0, CHUNK)],
                        dinvv.at[pl.ds(0, CHUNK)])
        pltpu.sync_copy(bidx_hbm.at[pl.ds(r0, CHUNK)],
                        bidxv.at[pl.ds(0, CHUNK)])

        def row_body(i, _):
            d = dinvv[pl.ds(i, 16)][0]
            b = bidxv[pl.ds(i, 16)][0]
            pcnt[b, pl.ds(0, 16)] = pcnt[b, pl.ds(0, 16)] + ones
            for k in range(HD // 16):
                sl = pl.ds(k * 16, 16)
                v = (arow0[i, sl] + arow1[i, sl] + zrow[i, sl]) * d
                v = jnp.maximum(v + biasv[sl], 0.0)
                psum[b, sl] = psum[b, sl] + v
                pmax[b, sl] = jnp.maximum(pmax[b, sl], v)
            return 0

        lax.fori_loop(0, CHUNK, row_body, 0)
        return 0

    lax.fori_loop(0, nj, chunk_body, 0)
    pltpu.sync_copy(psum, psum_hbm.at[c, s])
    pltpu.sync_copy(pmax, pmax_hbm.at[c, s])
    pltpu.sync_copy(pcnt, pcnt_hbm.at[c, s])


def _final_body(psum_ref, pmax_ref, pcnt_ref, wa_ref, ba_ref,
                out_ref, aggr_ref):
    sums = jnp.sum(psum_ref[...], axis=(0, 1))
    maxs = jnp.max(pmax_ref[...], axis=(0, 1))
    cnt = jnp.sum(pcnt_ref[...], axis=(0, 1, 3)) / 16.0
    scale = 1.0 / jnp.maximum(cnt, 1.0)[:, None]
    aggr = jnp.concatenate([sums * scale, maxs], axis=1)
    out = jnp.dot(aggr, wa_ref[...], preferred_element_type=_f32)
    out_ref[...] = out + ba_ref[...]
    aggr_ref[...] = aggr


def _final(psum, pmax, pcnt, wa, ba):
    return pl.pallas_call(
        _final_body,
        out_shape=[
            jax.ShapeDtypeStruct((G, CO), _f32),
            jax.ShapeDtypeStruct((G, 2 * HD), _f32),
        ],
    )(psum, pmax, pcnt, wa, ba.reshape(1, CO))


def kernel(x, edge_index, batch_index, W1, b1, W2, b2, W3, b3, Wa, ba):
    src = edge_index[0].reshape(NW, RPT // 25, 25, CHUNK)
    dst = edge_index[1].reshape(NW, RPT // 25, 25, CHUNK)
    dst_deg = edge_index[1].reshape(NW, RPT, CHUNK)

    deg16 = _deg_kernel(dst_deg)
    z1, dinv = _tc0(deg16, x, W1)
    dinv1 = dinv.reshape(-1)

    acc1 = _sc_scatter(src, dst, z1)
    z2 = _tcmid(acc1, z1, dinv, b1, W2)
    acc2 = _sc_scatter(src, dst, z2)
    z3 = _tcmid(acc2, z2, dinv, b2, W3)
    acc3 = _sc_scatter(src, dst, z3)
    psum, pmax, pcnt = _sc_pool(acc3, z3, dinv1, batch_index, b3)

    out, aggr = _final(psum, pmax, pcnt, Wa, ba)
    return (out, aggr)
